# Initial kernel scaffold; baseline (speedup 1.0000x reference)
import jax
import jax.numpy as jnp
from jax.experimental import pallas as pl


def kernel(phoneme_ids, table):
    raise NotImplementedError("write your pallas kernel here")


# speedup vs baseline: 1.5903x; 1.5903x over previous
import functools

import jax
import jax.numpy as jnp
from jax import lax
from jax.experimental import pallas as pl
from jax.experimental.pallas import tpu as pltpu
from jax.experimental.pallas import tpu_sc as plsc

D = 1024
NC = 2
NS = 16
NW = NC * NS
B_TOT = 1024 * 200
B_PER_W = B_TOT // NW
C = 40
NCHUNK = B_PER_W // C


def _sc_gather(ids_flat, table):
    mesh = plsc.VectorSubcoreMesh(core_axis_name="c", subcore_axis_name="s")

    @functools.partial(
        pl.kernel,
        mesh=mesh,
        out_type=jax.ShapeDtypeStruct((B_TOT, D), jnp.float32),
        scratch_types=[
            pltpu.VMEM((B_PER_W,), jnp.int32),
            pltpu.VMEM((C, D), jnp.float32),
            pltpu.VMEM((C, D), jnp.float32),
            pltpu.SemaphoreType.DMA,
            pltpu.SemaphoreType.DMA,
            pltpu.SemaphoreType.DMA,
            pltpu.SemaphoreType.DMA,
        ],
    )
    def k(ids_hbm, table_hbm, out_hbm, idx_v, buf0, buf1, g0, g1, s0, s1):
        wid = lax.axis_index("s") * NC + lax.axis_index("c")
        base = pl.multiple_of(wid * B_PER_W, 8)
        pltpu.sync_copy(ids_hbm.at[pl.ds(base, B_PER_W)], idx_v)

        bufs = (buf0, buf1)
        gsems = (g0, g1)
        ssems = (s0, s1)

        def g_start(chunk, buf, sem):
            off = pl.multiple_of(chunk * C, 8)
            pltpu.async_copy(table_hbm.at[idx_v.at[pl.ds(off, C)]], buf, sem)

        def g_wait(buf, sem):
            pltpu.make_async_copy(table_hbm.at[pl.ds(0, C)], buf, sem).wait()

        def s_start(chunk, buf, sem):
            row = pl.multiple_of(base + chunk * C, 8)
            return pltpu.async_copy(buf, out_hbm.at[pl.ds(row, C)], sem)

        g_start(0, buf0, g0)
        g_start(1, buf1, g1)

        def pair(p, carry):
            for b in range(2):
                chunk = p * 2 + b
                g_wait(bufs[b], gsems[b])
                cp = s_start(chunk, bufs[b], ssems[b])
                cp.wait()

                @pl.when(chunk + 2 < NCHUNK)
                def _():
                    g_start(chunk + 2, bufs[b], gsems[b])

            return carry

        lax.fori_loop(0, NCHUNK // 2, pair, 0)

    return k(ids_flat, table)


def kernel(phoneme_ids, table):
    ids_flat = phoneme_ids.reshape(-1)
    out = _sc_gather(ids_flat, table)
    return out.reshape(phoneme_ids.shape[0], phoneme_ids.shape[1], D)
